# baseline (speedup 1.0000x reference)
import functools

import jax
import jax.numpy as jnp
from jax import lax
from jax.experimental import pallas as pl
from jax.experimental.pallas import tpu as pltpu
from jax.experimental.pallas import tpu_sc as plsc

H = 300
HP = 384
HC = 304
NJC = HC // 16
NJ_PAD = HP // 16
G3 = 3 * HP
A = 10001
AP = 10240
B = 160001
BP = 163840
MAX_NB = 16
NM = 500
T = 20
NW = 32
AW = AP // NW
BW = BP // NW
CA = 8
CB = 80


def _mesh():
    return plsc.VectorSubcoreMesh(core_axis_name="c", subcore_axis_name="s")


AT0, AT1 = 432, 208
BT0, BT1 = 6240, 4000


def _sc_atom_agg(msg_bond, a2b_flat, msg_atom):
    @functools.partial(
        pl.kernel,
        out_type=[
            jax.ShapeDtypeStruct((AP, HP), jnp.float32),
            jax.ShapeDtypeStruct((AP, HP), jnp.float32),
        ],
        mesh=_mesh(),
        scratch_types=[
            pltpu.VMEM((2, CA * MAX_NB), jnp.int32),
            pltpu.VMEM((2, CA * MAX_NB, HP), jnp.float32),
            pltpu.VMEM((CA, HP), jnp.float32),
            pltpu.VMEM((CA, HP), jnp.float32),
            pltpu.VMEM((CA, HP), jnp.float32),
            pltpu.SemaphoreType.DMA,
            pltpu.SemaphoreType.DMA,
        ],
    )
    def k(mb_hbm, a2b_hbm, ma_hbm, upd_hbm, agg_hbm,
          idx_v, nei_v, ain_v, upd_v, agg_v, sem0, sem1):
        c = lax.axis_index("c")
        s = lax.axis_index("s")
        base0 = jnp.where(c == 0, s * AT0, 16 * AT0 + s * AT1)
        nch = jnp.where(c == 0, AT0 // CA, AT1 // CA)
        sems = (sem0, sem1)

        zero = jnp.zeros((16,), jnp.float32)
        for a in range(CA):
            for j in range(NJC, NJ_PAD):
                upd_v[a, pl.ds(j * 16, 16)] = zero
                agg_v[a, pl.ds(j * 16, 16)] = zero

        def issue(ci, b):
            abase = base0 + ci * CA
            pltpu.sync_copy(a2b_hbm.at[pl.ds(abase * MAX_NB, CA * MAX_NB)],
                            idx_v.at[b])
            pltpu.async_copy(mb_hbm.at[idx_v.at[b]], nei_v.at[b], sems[b])

        def wait(b):
            pltpu.make_async_copy(mb_hbm.at[idx_v.at[b]], nei_v.at[b],
                                  sems[b]).wait()

        issue(0, 0)

        def outer(i, _):
            ci0 = i * 2
            for b in range(2):
                ci = ci0 + b
                nxt = ci + 1

                @pl.when(nxt < nch)
                def _():
                    issue(nxt, 1 - b)

                wait(b)
                abase = base0 + ci * CA
                pltpu.sync_copy(ma_hbm.at[pl.ds(abase, CA)], ain_v)

                def per_j(j, _):
                    col = j * 16
                    for a in range(CA):
                        v = [nei_v[b, a * MAX_NB + kk, pl.ds(col, 16)]
                             for kk in range(MAX_NB)]
                        m = v[0]
                        for kk in range(1, MAX_NB):
                            m = jnp.maximum(m, v[kk])
                        s = v
                        while len(s) > 1:
                            h = len(s) // 2
                            s = [s[i2] + s[h + i2] for i2 in range(h)]
                        g = s[0] * m
                        agg_v[a, pl.ds(col, 16)] = g
                        upd_v[a, pl.ds(col, 16)] = (ain_v[a, pl.ds(col, 16)]
                                                    + g)
                    return 0

                lax.fori_loop(0, NJC, per_j, 0)
                pltpu.sync_copy(agg_v, agg_hbm.at[pl.ds(abase, CA)])
                pltpu.sync_copy(upd_v, upd_hbm.at[pl.ds(abase, CA)])
            return 0

        lax.fori_loop(0, nch // 2, outer, 0)

    return k(msg_bond, a2b_flat, msg_atom)


def _sc_bond_gather(msg_atom, msg_bond, b2a, b2revb):
    @functools.partial(
        pl.kernel,
        out_type=jax.ShapeDtypeStruct((BP, HP), jnp.float32),
        mesh=_mesh(),
        scratch_types=[
            pltpu.VMEM((2, CB), jnp.int32),
            pltpu.VMEM((2, CB), jnp.int32),
            pltpu.VMEM((2, CB, HP), jnp.float32),
            pltpu.VMEM((2, CB, HP), jnp.float32),
            pltpu.SemaphoreType.DMA,
            pltpu.SemaphoreType.DMA,
            pltpu.SemaphoreType.DMA,
            pltpu.SemaphoreType.DMA,
        ],
    )
    def k(ma_hbm, mb_hbm, b2a_hbm, b2revb_hbm, pre_hbm,
          idx1_v, idx2_v, g1_v, g2_v, s1a, s1b, s2a, s2b):
        c = lax.axis_index("c")
        s = lax.axis_index("s")
        base0 = jnp.where(c == 0, s * BT0, 16 * BT0 + s * BT1)
        nch = jnp.where(c == 0, BT0 // CB, BT1 // CB)
        sems1 = (s1a, s1b)
        sems2 = (s2a, s2b)

        def issue(ci, b):
            bbase = base0 + ci * CB
            pltpu.sync_copy(b2a_hbm.at[pl.ds(bbase, CB)], idx1_v.at[b])
            pltpu.sync_copy(b2revb_hbm.at[pl.ds(bbase, CB)], idx2_v.at[b])
            pltpu.async_copy(ma_hbm.at[idx1_v.at[b]], g1_v.at[b], sems1[b])
            pltpu.async_copy(mb_hbm.at[idx2_v.at[b]], g2_v.at[b], sems2[b])

        def wait(b):
            pltpu.make_async_copy(ma_hbm.at[idx1_v.at[b]], g1_v.at[b],
                                  sems1[b]).wait()
            pltpu.make_async_copy(mb_hbm.at[idx2_v.at[b]], g2_v.at[b],
                                  sems2[b]).wait()

        issue(0, 0)

        def outer(i, _):
            ci0 = i * 2
            for b in range(2):
                ci = ci0 + b
                nxt = ci + 1

                @pl.when(nxt < nch)
                def _():
                    issue(nxt, 1 - b)

                wait(b)
                bbase = base0 + ci * CB

                def per_r(r4, _):
                    for u in range(4):
                        r = r4 * 4 + u
                        for j in range(NJC):
                            col = j * 16
                            g1_v[b, r, pl.ds(col, 16)] = (
                                g1_v[b, r, pl.ds(col, 16)]
                                - g2_v[b, r, pl.ds(col, 16)])
                    return 0

                lax.fori_loop(0, CB // 4, per_r, 0)
                pltpu.sync_copy(g1_v.at[b], pre_hbm.at[pl.ds(bbase, CB)])
            return 0

        lax.fori_loop(0, nch // 2, outer, 0)

    return k(msg_atom, msg_bond, b2a, b2revb)


def _tc_embed(x, wt, rows_out, rb):
    rows_in, f = x.shape
    grid = (rows_in + rb - 1) // rb

    def body(x_ref, w_ref, o_ref):
        o_ref[...] = jnp.maximum(
            jnp.dot(x_ref[...], w_ref[...],
                    preferred_element_type=jnp.float32), 0.0)

    return pl.pallas_call(
        body,
        grid=(grid,),
        in_specs=[
            pl.BlockSpec((rb, f), lambda i: (i, 0)),
            pl.BlockSpec((f, HP), lambda i: (0, 0)),
        ],
        out_specs=pl.BlockSpec((rb, HP), lambda i: (i, 0)),
        out_shape=jax.ShapeDtypeStruct((rows_out, HP), jnp.float32),
    )(x, wt)


def _tc_embed_t(xt, wt, rows_out, rb):
    f, rows_in = xt.shape
    grid = (rows_in + rb - 1) // rb

    def body(x_ref, w_ref, o_ref):
        o_ref[...] = jnp.maximum(
            lax.dot_general(x_ref[...], w_ref[...],
                            (((0,), (0,)), ((), ())),
                            preferred_element_type=jnp.float32), 0.0)

    return pl.pallas_call(
        body,
        grid=(grid,),
        in_specs=[
            pl.BlockSpec((f, rb), lambda i: (0, i)),
            pl.BlockSpec((f, HP), lambda i: (0, 0)),
        ],
        out_specs=pl.BlockSpec((rb, HP), lambda i: (i, 0)),
        out_shape=jax.ShapeDtypeStruct((rows_out, HP), jnp.float32),
    )(xt, wt)


def _tc_bond_update(pre, wt, input_bond):
    rb = 512

    def body(p_ref, w_ref, ib_ref, o_ref):
        o_ref[...] = jnp.maximum(
            ib_ref[...] + jnp.dot(p_ref[...], w_ref[...],
                                  preferred_element_type=jnp.float32), 0.0)

    return pl.pallas_call(
        body,
        grid=((B + rb - 1) // rb,),
        in_specs=[
            pl.BlockSpec((rb, HP), lambda i: (i, 0)),
            pl.BlockSpec((HP, HP), lambda i: (0, 0)),
            pl.BlockSpec((rb, HP), lambda i: (i, 0)),
        ],
        out_specs=pl.BlockSpec((rb, HP), lambda i: (i, 0)),
        out_shape=jax.ShapeDtypeStruct((BP, HP), jnp.float32),
    )(pre, wt, input_bond)


def _tc_lr(agg, ma, ia, b0, b1, b2):
    rb = 512

    def body(a_ref, m_ref, i_ref, b0_ref, b1_ref, b2_ref, o_ref):
        acc = jnp.dot(a_ref[...], b0_ref[...],
                      preferred_element_type=jnp.float32)
        acc += jnp.dot(m_ref[...], b1_ref[...],
                       preferred_element_type=jnp.float32)
        acc += jnp.dot(i_ref[...], b2_ref[...],
                       preferred_element_type=jnp.float32)
        o_ref[...] = acc

    return pl.pallas_call(
        body,
        grid=(AP // rb,),
        in_specs=[
            pl.BlockSpec((rb, HP), lambda i: (i, 0)),
            pl.BlockSpec((rb, HP), lambda i: (i, 0)),
            pl.BlockSpec((rb, HP), lambda i: (i, 0)),
            pl.BlockSpec((HP, HP), lambda i: (0, 0)),
            pl.BlockSpec((HP, HP), lambda i: (0, 0)),
            pl.BlockSpec((HP, HP), lambda i: (0, 0)),
        ],
        out_specs=pl.BlockSpec((rb, HP), lambda i: (i, 0)),
        out_shape=jax.ShapeDtypeStruct((AP, HP), jnp.float32),
    )(agg, ma, ia, b0, b1, b2)


def _tc_gru(hid_am, gbias, wihf, bihf, whhf, bhhf, wihb, bibh, whhb, bhhb,
            wo1, wo2, wob):
    def body(hid_ref, gb_ref, wihf_ref, bihf_ref, whhf_ref, bhhf_ref,
             wihb_ref, bibh_ref, whhb_ref, bhhb_ref,
             wo1_ref, wo2_ref, wob_ref, o_ref, hf_buf):
        h0 = hid_ref[:, 0, :]
        for t in range(1, T):
            h0 = jnp.maximum(h0, hid_ref[:, t, :])
        h0 = h0[:, 0:H]

        gb = gb_ref[...]

        def gru_step(x, h, wih_ref, bih_ref, whh_ref, bhh_ref):
            gi = jnp.dot(x, wih_ref[...],
                         preferred_element_type=jnp.float32) + bih_ref[...]
            gh = jnp.dot(h, whh_ref[...],
                         preferred_element_type=jnp.float32) + bhh_ref[...]
            r = jax.nn.sigmoid(gi[:, 0:H] + gh[:, 0:H])
            z = jax.nn.sigmoid(gi[:, H:2 * H] + gh[:, H:2 * H])
            n = jnp.tanh(gi[:, 2 * H:3 * H] + r * gh[:, 2 * H:3 * H])
            return (1.0 - z) * n + z * h

        def fwd_step(t, hf):
            xf = jnp.maximum(hid_ref[:, t, :] + gb, 0.0)
            hf = gru_step(xf, hf, wihf_ref, bihf_ref, whhf_ref, bhhf_ref)
            hf_buf[t] = hf
            return hf

        lax.fori_loop(0, T, fwd_step, h0)

        def bwd_step(i, carry):
            hb, acc = carry
            t = T - 1 - i
            xb = jnp.maximum(hid_ref[:, t, :] + gb, 0.0)
            hb = gru_step(xb, hb, wihb_ref, bibh_ref, whhb_ref, bhhb_ref)
            ah = jnp.dot(hf_buf[t], wo1_ref[...],
                         preferred_element_type=jnp.float32)
            ah += jnp.dot(hb, wo2_ref[...],
                          preferred_element_type=jnp.float32)
            ah = jnp.maximum(ah + wob_ref[...], 0.0)
            return (hb, acc + ah)

        _, acc = lax.fori_loop(0, T, bwd_step,
                               (h0, jnp.zeros((NM, H), jnp.float32)))
        o_ref[...] = acc * (1.0 / T)

    return pl.pallas_call(
        body,
        out_shape=jax.ShapeDtypeStruct((NM, H), jnp.float32),
        scratch_shapes=[
            pltpu.VMEM((T, NM, H), jnp.float32),
        ],
    )(hid_am, gbias, wihf, bihf, whhf, bhhf, wihb, bibh, whhb, bhhb,
      wo1, wo2, wob)


def _pad_wt(w):
    return jnp.pad(w.T, ((0, 0), (0, HP - H)))


def _pad_sq(w):
    return jnp.pad(w.T, ((0, HP - H), (0, HP - H)))


def _pad_gates(w):
    wt = w.T
    out = jnp.zeros((HP, G3), jnp.float32)
    for g in range(3):
        out = lax.dynamic_update_slice(
            out, jnp.pad(wt[:, g * H:(g + 1) * H], ((0, HP - H), (0, 0))),
            (0, g * HP))
    return out


def _pad_gate_bias(b):
    out = jnp.zeros((1, G3), jnp.float32)
    for g in range(3):
        out = lax.dynamic_update_slice(out, b[g * H:(g + 1) * H][None, :],
                                       (0, g * HP))
    return out


def kernel(f_atoms, f_bonds, a2b, b2a, b2revb, a_scope, W_i_atom, W_i_bond,
           W_h_0, W_h_1, lr_W, gru_bias, gru_W_ih_f, gru_W_hh_f, gru_b_ih_f,
           gru_b_hh_f, gru_W_ih_b, gru_W_hh_b, gru_b_ih_b, gru_b_hh_b,
           W_o_W, W_o_b):
    a2b_flat = jnp.pad(a2b, ((0, AP - A), (0, 0))).reshape(-1)
    b2a_p = jnp.pad(b2a, (0, BP - B))
    b2revb_p = jnp.pad(b2revb, (0, BP - B))

    wia = _pad_wt(W_i_atom)
    wib = _pad_wt(W_i_bond)
    wh = [_pad_sq(W_h_0), _pad_sq(W_h_1)]
    lrt = lr_W.T
    b0 = jnp.pad(lrt[0:H], ((0, HP - H), (0, HP - H)))
    b1 = jnp.pad(lrt[H:2 * H], ((0, HP - H), (0, HP - H)))
    b2 = jnp.pad(lrt[2 * H:3 * H], ((0, HP - H), (0, HP - H)))
    gb = jnp.pad(gru_bias, (0, HP - H))[None, :]
    wihf = jnp.pad(gru_W_ih_f.T, ((0, HP - H), (0, 0)))
    whhf = gru_W_hh_f.T
    bihf = gru_b_ih_f[None, :]
    bhhf = gru_b_hh_f[None, :]
    wihb = jnp.pad(gru_W_ih_b.T, ((0, HP - H), (0, 0)))
    whhb = gru_W_hh_b.T
    bibh = gru_b_ih_b[None, :]
    bhhb = gru_b_hh_b[None, :]
    wot = W_o_W.T
    wo1 = wot[0:H]
    wo2 = wot[H:2 * H]
    wob = W_o_b[None, :]

    input_atom = _tc_embed(f_atoms, wia, AP, 512)
    input_bond = _tc_embed_t(f_bonds.T, wib, BP, 512)

    msg_atom = input_atom
    msg_bond = input_bond
    for it in range(2):
        msg_atom, _ = _sc_atom_agg(msg_bond, a2b_flat, msg_atom)
        pre = _sc_bond_gather(msg_atom, msg_bond, b2a_p, b2revb_p)
        msg_bond = _tc_bond_update(pre, wh[it], input_bond)

    _, agg = _sc_atom_agg(msg_bond, a2b_flat, msg_atom)
    hidden = _tc_lr(agg, msg_atom, input_atom, b0, b1, b2)

    hid_am = hidden[1:1 + NM * T].reshape(NM, T, HP)
    return _tc_gru(hid_am, gb, wihf, bihf, whhf, bhhf, wihb, bibh, whhb,
                   bhhb, wo1, wo2, wob)


# speedup vs baseline: 1.0756x; 1.0756x over previous
import functools

import jax
import jax.numpy as jnp
from jax import lax
from jax.experimental import pallas as pl
from jax.experimental.pallas import tpu as pltpu
from jax.experimental.pallas import tpu_sc as plsc

H = 300
HP = 384
HC = 304
NJC = HC // 16
NJ_PAD = HP // 16
G3 = 3 * HP
A = 10001
AP = 10240
B = 160001
BP = 163840
MAX_NB = 16
NM = 500
T = 20
NW = 32
AW = AP // NW
BW = BP // NW
CA = 8
CB = 80


def _mesh():
    return plsc.VectorSubcoreMesh(core_axis_name="c", subcore_axis_name="s")


AT0, AT1 = 432, 208
BT0, BT1 = 6240, 4000


def _sc_atom_agg(msg_bond, a2b_flat, msg_atom):
    @functools.partial(
        pl.kernel,
        out_type=[
            jax.ShapeDtypeStruct((AP, HP), jnp.float32),
            jax.ShapeDtypeStruct((AP, HP), jnp.float32),
        ],
        mesh=_mesh(),
        scratch_types=[
            pltpu.VMEM((2, CA * MAX_NB), jnp.int32),
            pltpu.VMEM((2, CA * MAX_NB, HP), jnp.float32),
            pltpu.VMEM((CA, HP), jnp.float32),
            pltpu.VMEM((CA, HP), jnp.float32),
            pltpu.VMEM((CA, HP), jnp.float32),
            pltpu.SemaphoreType.DMA,
            pltpu.SemaphoreType.DMA,
        ],
    )
    def k(mb_hbm, a2b_hbm, ma_hbm, upd_hbm, agg_hbm,
          idx_v, nei_v, ain_v, upd_v, agg_v, sem0, sem1):
        c = lax.axis_index("c")
        s = lax.axis_index("s")
        base0 = jnp.where(c == 0, s * AT0, 16 * AT0 + s * AT1)
        nch = jnp.where(c == 0, AT0 // CA, AT1 // CA)
        sems = (sem0, sem1)

        zero = jnp.zeros((16,), jnp.float32)
        for a in range(CA):
            for j in range(NJC, NJ_PAD):
                upd_v[a, pl.ds(j * 16, 16)] = zero
                agg_v[a, pl.ds(j * 16, 16)] = zero

        def issue(ci, b):
            abase = base0 + ci * CA
            pltpu.sync_copy(a2b_hbm.at[pl.ds(abase * MAX_NB, CA * MAX_NB)],
                            idx_v.at[b])
            pltpu.async_copy(mb_hbm.at[idx_v.at[b]], nei_v.at[b], sems[b])

        def wait(b):
            pltpu.make_async_copy(mb_hbm.at[idx_v.at[b]], nei_v.at[b],
                                  sems[b]).wait()

        issue(0, 0)

        def outer(i, _):
            ci0 = i * 2
            for b in range(2):
                ci = ci0 + b
                nxt = ci + 1

                @pl.when(nxt < nch)
                def _():
                    issue(nxt, 1 - b)

                wait(b)
                abase = base0 + ci * CA
                pltpu.sync_copy(ma_hbm.at[pl.ds(abase, CA)], ain_v)

                def per_j(j, _):
                    col = j * 16
                    for a in range(CA):
                        v = [nei_v[b, a * MAX_NB + kk, pl.ds(col, 16)]
                             for kk in range(MAX_NB)]
                        m = v[0]
                        for kk in range(1, MAX_NB):
                            m = jnp.maximum(m, v[kk])
                        s = v
                        while len(s) > 1:
                            h = len(s) // 2
                            s = [s[i2] + s[h + i2] for i2 in range(h)]
                        g = s[0] * m
                        agg_v[a, pl.ds(col, 16)] = g
                        upd_v[a, pl.ds(col, 16)] = (ain_v[a, pl.ds(col, 16)]
                                                    + g)
                    return 0

                lax.fori_loop(0, NJC, per_j, 0)
                pltpu.sync_copy(agg_v, agg_hbm.at[pl.ds(abase, CA)])
                pltpu.sync_copy(upd_v, upd_hbm.at[pl.ds(abase, CA)])
            return 0

        lax.fori_loop(0, nch // 2, outer, 0)

    return k(msg_bond, a2b_flat, msg_atom)


def _sc_bond_gather(msg_atom, msg_bond, b2a, b2revb):
    @functools.partial(
        pl.kernel,
        out_type=jax.ShapeDtypeStruct((BP, HP), jnp.float32),
        mesh=_mesh(),
        scratch_types=[
            pltpu.VMEM((2, CB), jnp.int32),
            pltpu.VMEM((2, CB), jnp.int32),
            pltpu.VMEM((2, CB, HP), jnp.float32),
            pltpu.VMEM((2, CB, HP), jnp.float32),
            pltpu.SemaphoreType.DMA,
            pltpu.SemaphoreType.DMA,
            pltpu.SemaphoreType.DMA,
            pltpu.SemaphoreType.DMA,
        ],
    )
    def k(ma_hbm, mb_hbm, b2a_hbm, b2revb_hbm, pre_hbm,
          idx1_v, idx2_v, g1_v, g2_v, s1a, s1b, s2a, s2b):
        c = lax.axis_index("c")
        s = lax.axis_index("s")
        base0 = jnp.where(c == 0, s * BT0, 16 * BT0 + s * BT1)
        nch = jnp.where(c == 0, BT0 // CB, BT1 // CB)
        sems1 = (s1a, s1b)
        sems2 = (s2a, s2b)

        def issue(ci, b):
            bbase = base0 + ci * CB
            pltpu.sync_copy(b2a_hbm.at[pl.ds(bbase, CB)], idx1_v.at[b])
            pltpu.sync_copy(b2revb_hbm.at[pl.ds(bbase, CB)], idx2_v.at[b])
            pltpu.async_copy(ma_hbm.at[idx1_v.at[b]], g1_v.at[b], sems1[b])
            pltpu.async_copy(mb_hbm.at[idx2_v.at[b]], g2_v.at[b], sems2[b])

        def wait(b):
            pltpu.make_async_copy(ma_hbm.at[idx1_v.at[b]], g1_v.at[b],
                                  sems1[b]).wait()
            pltpu.make_async_copy(mb_hbm.at[idx2_v.at[b]], g2_v.at[b],
                                  sems2[b]).wait()

        issue(0, 0)

        def outer(i, _):
            ci0 = i * 2
            for b in range(2):
                ci = ci0 + b
                nxt = ci + 1

                @pl.when(nxt < nch)
                def _():
                    issue(nxt, 1 - b)

                wait(b)
                bbase = base0 + ci * CB

                def per_r(r4, _):
                    for u in range(4):
                        r = r4 * 4 + u
                        for j in range(NJC):
                            col = j * 16
                            g1_v[b, r, pl.ds(col, 16)] = (
                                g1_v[b, r, pl.ds(col, 16)]
                                - g2_v[b, r, pl.ds(col, 16)])
                    return 0

                lax.fori_loop(0, CB // 4, per_r, 0)
                pltpu.sync_copy(g1_v.at[b], pre_hbm.at[pl.ds(bbase, CB)])
            return 0

        lax.fori_loop(0, nch // 2, outer, 0)

    return k(msg_atom, msg_bond, b2a, b2revb)


def _tc_embed(x, wt, rows_out, rb):
    rows_in, f = x.shape
    grid = (rows_in + rb - 1) // rb

    def body(x_ref, w_ref, o_ref):
        o_ref[...] = jnp.maximum(
            jnp.dot(x_ref[...], w_ref[...],
                    preferred_element_type=jnp.float32), 0.0)

    return pl.pallas_call(
        body,
        grid=(grid,),
        in_specs=[
            pl.BlockSpec((rb, f), lambda i: (i, 0)),
            pl.BlockSpec((f, HP), lambda i: (0, 0)),
        ],
        out_specs=pl.BlockSpec((rb, HP), lambda i: (i, 0)),
        out_shape=jax.ShapeDtypeStruct((rows_out, HP), jnp.float32),
    )(x, wt)


def _tc_embed_t(xt, wt, rows_out, rb):
    f, rows_in = xt.shape
    grid = (rows_in + rb - 1) // rb

    def body(x_ref, w_ref, o_ref):
        o_ref[...] = jnp.maximum(
            lax.dot_general(x_ref[...], w_ref[...],
                            (((0,), (0,)), ((), ())),
                            preferred_element_type=jnp.float32), 0.0)

    return pl.pallas_call(
        body,
        grid=(grid,),
        in_specs=[
            pl.BlockSpec((f, rb), lambda i: (0, i)),
            pl.BlockSpec((f, HP), lambda i: (0, 0)),
        ],
        out_specs=pl.BlockSpec((rb, HP), lambda i: (i, 0)),
        out_shape=jax.ShapeDtypeStruct((rows_out, HP), jnp.float32),
    )(xt, wt)


def _tc_bond_update(pre, wt, input_bond):
    rb = 1024

    def body(p_ref, w_ref, ib_ref, o_ref):
        o_ref[...] = jnp.maximum(
            ib_ref[...] + jnp.dot(p_ref[...], w_ref[...],
                                  preferred_element_type=jnp.float32), 0.0)

    return pl.pallas_call(
        body,
        grid=((B + rb - 1) // rb,),
        in_specs=[
            pl.BlockSpec((rb, HP), lambda i: (i, 0)),
            pl.BlockSpec((HP, HP), lambda i: (0, 0)),
            pl.BlockSpec((rb, HP), lambda i: (i, 0)),
        ],
        out_specs=pl.BlockSpec((rb, HP), lambda i: (i, 0)),
        out_shape=jax.ShapeDtypeStruct((BP, HP), jnp.float32),
    )(pre, wt, input_bond)


def _tc_lr(agg, ma, ia, b0, b1, b2):
    rb = 512

    def body(a_ref, m_ref, i_ref, b0_ref, b1_ref, b2_ref, o_ref):
        acc = jnp.dot(a_ref[...], b0_ref[...],
                      preferred_element_type=jnp.float32)
        acc += jnp.dot(m_ref[...], b1_ref[...],
                       preferred_element_type=jnp.float32)
        acc += jnp.dot(i_ref[...], b2_ref[...],
                       preferred_element_type=jnp.float32)
        o_ref[...] = acc

    return pl.pallas_call(
        body,
        grid=(AP // rb,),
        in_specs=[
            pl.BlockSpec((rb, HP), lambda i: (i, 0)),
            pl.BlockSpec((rb, HP), lambda i: (i, 0)),
            pl.BlockSpec((rb, HP), lambda i: (i, 0)),
            pl.BlockSpec((HP, HP), lambda i: (0, 0)),
            pl.BlockSpec((HP, HP), lambda i: (0, 0)),
            pl.BlockSpec((HP, HP), lambda i: (0, 0)),
        ],
        out_specs=pl.BlockSpec((rb, HP), lambda i: (i, 0)),
        out_shape=jax.ShapeDtypeStruct((AP, HP), jnp.float32),
    )(agg, ma, ia, b0, b1, b2)


def _tc_gru(hid_am, gbias, wihf, bihf, whhf, bhhf, wihb, bibh, whhb, bhhb,
            wo1, wo2, wob):
    def body(hid_ref, gb_ref, wihf_ref, bihf_ref, whhf_ref, bhhf_ref,
             wihb_ref, bibh_ref, whhb_ref, bhhb_ref,
             wo1_ref, wo2_ref, wob_ref, o_ref, hf_buf):
        h0 = hid_ref[:, 0, :]
        for t in range(1, T):
            h0 = jnp.maximum(h0, hid_ref[:, t, :])

        gb = gb_ref[...]

        def gru_step(x, h, wih_ref, bih_ref, whh_ref, bhh_ref):
            gi = jnp.dot(x, wih_ref[...],
                         preferred_element_type=jnp.float32) + bih_ref[...]
            gh = jnp.dot(h, whh_ref[...],
                         preferred_element_type=jnp.float32) + bhh_ref[...]
            r = jax.nn.sigmoid(gi[:, 0:HP] + gh[:, 0:HP])
            z = jax.nn.sigmoid(gi[:, HP:2 * HP] + gh[:, HP:2 * HP])
            n = jnp.tanh(gi[:, 2 * HP:3 * HP] + r * gh[:, 2 * HP:3 * HP])
            return (1.0 - z) * n + z * h

        def fwd_step(t, hf):
            xf = jnp.maximum(hid_ref[:, t, :] + gb, 0.0)
            hf = gru_step(xf, hf, wihf_ref, bihf_ref, whhf_ref, bhhf_ref)
            hf_buf[t] = hf
            return hf

        lax.fori_loop(0, T, fwd_step, h0)

        def bwd_step(i, carry):
            hb, acc = carry
            t = T - 1 - i
            xb = jnp.maximum(hid_ref[:, t, :] + gb, 0.0)
            hb = gru_step(xb, hb, wihb_ref, bibh_ref, whhb_ref, bhhb_ref)
            ah = jnp.dot(hf_buf[t], wo1_ref[...],
                         preferred_element_type=jnp.float32)
            ah += jnp.dot(hb, wo2_ref[...],
                          preferred_element_type=jnp.float32)
            ah = jnp.maximum(ah + wob_ref[...], 0.0)
            return (hb, acc + ah)

        _, acc = lax.fori_loop(0, T, bwd_step,
                               (h0, jnp.zeros((NM, HP), jnp.float32)))
        o_ref[...] = acc * (1.0 / T)

    return pl.pallas_call(
        body,
        out_shape=jax.ShapeDtypeStruct((NM, HP), jnp.float32),
        scratch_shapes=[
            pltpu.VMEM((T, NM, HP), jnp.float32),
        ],
    )(hid_am, gbias, wihf, bihf, whhf, bhhf, wihb, bibh, whhb, bhhb,
      wo1, wo2, wob)


def _pad_wt(w):
    return jnp.pad(w.T, ((0, 0), (0, HP - H)))


def _pad_sq(w):
    return jnp.pad(w.T, ((0, HP - H), (0, HP - H)))


def _pad_gates(w):
    wt = w.T
    out = jnp.zeros((HP, G3), jnp.float32)
    for g in range(3):
        out = lax.dynamic_update_slice(
            out, jnp.pad(wt[:, g * H:(g + 1) * H], ((0, HP - H), (0, 0))),
            (0, g * HP))
    return out


def _pad_gate_bias(b):
    out = jnp.zeros((1, G3), jnp.float32)
    for g in range(3):
        out = lax.dynamic_update_slice(out, b[g * H:(g + 1) * H][None, :],
                                       (0, g * HP))
    return out


def kernel(f_atoms, f_bonds, a2b, b2a, b2revb, a_scope, W_i_atom, W_i_bond,
           W_h_0, W_h_1, lr_W, gru_bias, gru_W_ih_f, gru_W_hh_f, gru_b_ih_f,
           gru_b_hh_f, gru_W_ih_b, gru_W_hh_b, gru_b_ih_b, gru_b_hh_b,
           W_o_W, W_o_b):
    a2b_flat = jnp.pad(a2b, ((0, AP - A), (0, 0))).reshape(-1)
    b2a_p = jnp.pad(b2a, (0, BP - B))
    b2revb_p = jnp.pad(b2revb, (0, BP - B))

    wia = _pad_wt(W_i_atom)
    wib = _pad_wt(W_i_bond)
    wh = [_pad_sq(W_h_0), _pad_sq(W_h_1)]
    lrt = lr_W.T
    b0 = jnp.pad(lrt[0:H], ((0, HP - H), (0, HP - H)))
    b1 = jnp.pad(lrt[H:2 * H], ((0, HP - H), (0, HP - H)))
    b2 = jnp.pad(lrt[2 * H:3 * H], ((0, HP - H), (0, HP - H)))
    gb = jnp.pad(gru_bias, (0, HP - H))[None, :]
    wihf = _pad_gates(gru_W_ih_f)
    whhf = _pad_gates(gru_W_hh_f)
    bihf = _pad_gate_bias(gru_b_ih_f)
    bhhf = _pad_gate_bias(gru_b_hh_f)
    wihb = _pad_gates(gru_W_ih_b)
    whhb = _pad_gates(gru_W_hh_b)
    bibh = _pad_gate_bias(gru_b_ih_b)
    bhhb = _pad_gate_bias(gru_b_hh_b)
    wot = W_o_W.T
    wo1 = jnp.pad(wot[0:H], ((0, HP - H), (0, HP - H)))
    wo2 = jnp.pad(wot[H:2 * H], ((0, HP - H), (0, HP - H)))
    wob = jnp.pad(W_o_b, (0, HP - H))[None, :]

    input_atom = _tc_embed(f_atoms, wia, AP, 512)
    input_bond = _tc_embed_t(f_bonds.T, wib, BP, 512)

    msg_atom = input_atom
    msg_bond = input_bond
    for it in range(2):
        msg_atom, _ = _sc_atom_agg(msg_bond, a2b_flat, msg_atom)
        pre = _sc_bond_gather(msg_atom, msg_bond, b2a_p, b2revb_p)
        msg_bond = _tc_bond_update(pre, wh[it], input_bond)

    _, agg = _sc_atom_agg(msg_bond, a2b_flat, msg_atom)
    hidden = _tc_lr(agg, msg_atom, input_atom, b0, b1, b2)

    hid_am = hidden[1:1 + NM * T].reshape(NM, T, HP)
    mol = _tc_gru(hid_am, gb, wihf, bihf, whhf, bhhf, wihb, bibh, whhb, bhhb,
                  wo1, wo2, wob)
    return mol[:, :H]


# speedup vs baseline: 1.1133x; 1.0350x over previous
import functools

import jax
import jax.numpy as jnp
from jax import lax
from jax.experimental import pallas as pl
from jax.experimental.pallas import tpu as pltpu
from jax.experimental.pallas import tpu_sc as plsc

H = 300
HP = 384
HC = 304
NJC = HC // 16
NJ_PAD = HP // 16
G3 = 3 * HP
A = 10001
AP = 10240
B = 160001
BP = 163840
MAX_NB = 16
NM = 500
T = 20
NW = 32
AW = AP // NW
BW = BP // NW
CA = 8
CB = 80


def _mesh():
    return plsc.VectorSubcoreMesh(core_axis_name="c", subcore_axis_name="s")


AT0, AT1 = 432, 208
BT0, BT1 = 6240, 4000


def _sc_atom_agg(msg_bond, a2b_flat, msg_atom):
    @functools.partial(
        pl.kernel,
        out_type=[
            jax.ShapeDtypeStruct((AP, HP), jnp.float32),
            jax.ShapeDtypeStruct((AP, HP), jnp.float32),
        ],
        mesh=_mesh(),
        scratch_types=[
            pltpu.VMEM((2, CA * MAX_NB), jnp.int32),
            pltpu.VMEM((2, CA * MAX_NB, HP), jnp.float32),
            pltpu.VMEM((CA, HP), jnp.float32),
            pltpu.VMEM((CA, HP), jnp.float32),
            pltpu.VMEM((CA, HP), jnp.float32),
            pltpu.SemaphoreType.DMA,
            pltpu.SemaphoreType.DMA,
        ],
    )
    def k(mb_hbm, a2b_hbm, ma_hbm, upd_hbm, agg_hbm,
          idx_v, nei_v, ain_v, upd_v, agg_v, sem0, sem1):
        c = lax.axis_index("c")
        s = lax.axis_index("s")
        base0 = jnp.where(c == 0, s * AT0, 16 * AT0 + s * AT1)
        nch = jnp.where(c == 0, AT0 // CA, AT1 // CA)
        sems = (sem0, sem1)

        zero = jnp.zeros((16,), jnp.float32)
        for a in range(CA):
            for j in range(NJC, NJ_PAD):
                upd_v[a, pl.ds(j * 16, 16)] = zero
                agg_v[a, pl.ds(j * 16, 16)] = zero

        def issue(ci, b):
            abase = base0 + ci * CA
            pltpu.sync_copy(a2b_hbm.at[pl.ds(abase * MAX_NB, CA * MAX_NB)],
                            idx_v.at[b])
            pltpu.async_copy(mb_hbm.at[idx_v.at[b]], nei_v.at[b], sems[b])

        def wait(b):
            pltpu.make_async_copy(mb_hbm.at[idx_v.at[b]], nei_v.at[b],
                                  sems[b]).wait()

        issue(0, 0)

        def outer(i, _):
            ci0 = i * 2
            for b in range(2):
                ci = ci0 + b
                nxt = ci + 1

                @pl.when(nxt < nch)
                def _():
                    issue(nxt, 1 - b)

                wait(b)
                abase = base0 + ci * CA
                pltpu.sync_copy(ma_hbm.at[pl.ds(abase, CA)], ain_v)

                def per_j(j, _):
                    col = j * 16
                    for a in range(CA):
                        v = [nei_v[b, a * MAX_NB + kk, pl.ds(col, 16)]
                             for kk in range(MAX_NB)]
                        m = v[0]
                        for kk in range(1, MAX_NB):
                            m = jnp.maximum(m, v[kk])
                        s = v
                        while len(s) > 1:
                            h = len(s) // 2
                            s = [s[i2] + s[h + i2] for i2 in range(h)]
                        g = s[0] * m
                        agg_v[a, pl.ds(col, 16)] = g
                        upd_v[a, pl.ds(col, 16)] = (ain_v[a, pl.ds(col, 16)]
                                                    + g)
                    return 0

                lax.fori_loop(0, NJC, per_j, 0)
                pltpu.sync_copy(agg_v, agg_hbm.at[pl.ds(abase, CA)])
                pltpu.sync_copy(upd_v, upd_hbm.at[pl.ds(abase, CA)])
            return 0

        lax.fori_loop(0, nch // 2, outer, 0)

    return k(msg_bond, a2b_flat, msg_atom)


def _sc_bond_gather(msg_atom, msg_bond, b2a, b2revb):
    @functools.partial(
        pl.kernel,
        out_type=jax.ShapeDtypeStruct((BP, HP), jnp.float32),
        mesh=_mesh(),
        scratch_types=[
            pltpu.VMEM((2, CB), jnp.int32),
            pltpu.VMEM((2, CB), jnp.int32),
            pltpu.VMEM((2, CB, HP), jnp.float32),
            pltpu.VMEM((2, CB, HP), jnp.float32),
            pltpu.SemaphoreType.DMA,
            pltpu.SemaphoreType.DMA,
            pltpu.SemaphoreType.DMA,
            pltpu.SemaphoreType.DMA,
        ],
    )
    def k(ma_hbm, mb_hbm, b2a_hbm, b2revb_hbm, pre_hbm,
          idx1_v, idx2_v, g1_v, g2_v, s1a, s1b, s2a, s2b):
        c = lax.axis_index("c")
        s = lax.axis_index("s")
        base0 = jnp.where(c == 0, s * BT0, 16 * BT0 + s * BT1)
        nch = jnp.where(c == 0, BT0 // CB, BT1 // CB)
        sems1 = (s1a, s1b)
        sems2 = (s2a, s2b)

        def issue(ci, b):
            bbase = base0 + ci * CB
            pltpu.sync_copy(b2a_hbm.at[pl.ds(bbase, CB)], idx1_v.at[b])
            pltpu.sync_copy(b2revb_hbm.at[pl.ds(bbase, CB)], idx2_v.at[b])
            pltpu.async_copy(ma_hbm.at[idx1_v.at[b]], g1_v.at[b], sems1[b])
            pltpu.async_copy(mb_hbm.at[idx2_v.at[b]], g2_v.at[b], sems2[b])

        def wait(b):
            pltpu.make_async_copy(ma_hbm.at[idx1_v.at[b]], g1_v.at[b],
                                  sems1[b]).wait()
            pltpu.make_async_copy(mb_hbm.at[idx2_v.at[b]], g2_v.at[b],
                                  sems2[b]).wait()

        issue(0, 0)

        def outer(i, _):
            ci0 = i * 2
            for b in range(2):
                ci = ci0 + b
                nxt = ci + 1

                @pl.when(nxt < nch)
                def _():
                    issue(nxt, 1 - b)

                wait(b)
                bbase = base0 + ci * CB

                def per_r(r4, _):
                    for u in range(4):
                        r = r4 * 4 + u
                        for j in range(NJC):
                            col = j * 16
                            g1_v[b, r, pl.ds(col, 16)] = (
                                g1_v[b, r, pl.ds(col, 16)]
                                - g2_v[b, r, pl.ds(col, 16)])
                    return 0

                lax.fori_loop(0, CB // 4, per_r, 0)
                pltpu.sync_copy(g1_v.at[b], pre_hbm.at[pl.ds(bbase, CB)])
            return 0

        lax.fori_loop(0, nch // 2, outer, 0)

    return k(msg_atom, msg_bond, b2a, b2revb)


def _tc_embed(x, wt, rows_out, rb):
    rows_in, f = x.shape
    grid = (rows_in + rb - 1) // rb

    def body(x_ref, w_ref, o_ref):
        o_ref[...] = jnp.maximum(
            jnp.dot(x_ref[...], w_ref[...],
                    preferred_element_type=jnp.float32), 0.0)

    return pl.pallas_call(
        body,
        grid=(grid,),
        in_specs=[
            pl.BlockSpec((rb, f), lambda i: (i, 0)),
            pl.BlockSpec((f, HP), lambda i: (0, 0)),
        ],
        out_specs=pl.BlockSpec((rb, HP), lambda i: (i, 0)),
        out_shape=jax.ShapeDtypeStruct((rows_out, HP), jnp.float32),
    )(x, wt)


def _tc_embed_t(xt, wt, rows_out, rb):
    f, rows_in = xt.shape
    grid = (rows_in + rb - 1) // rb

    def body(x_ref, w_ref, o_ref):
        o_ref[...] = jnp.maximum(
            lax.dot_general(x_ref[...], w_ref[...],
                            (((0,), (0,)), ((), ())),
                            preferred_element_type=jnp.float32), 0.0)

    return pl.pallas_call(
        body,
        grid=(grid,),
        in_specs=[
            pl.BlockSpec((f, rb), lambda i: (0, i)),
            pl.BlockSpec((f, HP), lambda i: (0, 0)),
        ],
        out_specs=pl.BlockSpec((rb, HP), lambda i: (i, 0)),
        out_shape=jax.ShapeDtypeStruct((rows_out, HP), jnp.float32),
    )(xt, wt)


def _tc_bond_update(pre, wt, input_bond):
    rb = 1024

    def body(p_ref, w_ref, ib_ref, o_ref):
        o_ref[...] = jnp.maximum(
            ib_ref[...] + jnp.dot(p_ref[...], w_ref[...],
                                  preferred_element_type=jnp.float32), 0.0)

    return pl.pallas_call(
        body,
        grid=((B + rb - 1) // rb,),
        in_specs=[
            pl.BlockSpec((rb, HP), lambda i: (i, 0)),
            pl.BlockSpec((HP, HP), lambda i: (0, 0)),
            pl.BlockSpec((rb, HP), lambda i: (i, 0)),
        ],
        out_specs=pl.BlockSpec((rb, HP), lambda i: (i, 0)),
        out_shape=jax.ShapeDtypeStruct((BP, HP), jnp.float32),
    )(pre, wt, input_bond)


def _tc_lr(agg, ma, ia, b0, b1, b2):
    rb = 1024

    def body(a_ref, m_ref, i_ref, b0_ref, b1_ref, b2_ref, o_ref):
        acc = jnp.dot(a_ref[...], b0_ref[...],
                      preferred_element_type=jnp.float32)
        acc += jnp.dot(m_ref[...], b1_ref[...],
                       preferred_element_type=jnp.float32)
        acc += jnp.dot(i_ref[...], b2_ref[...],
                       preferred_element_type=jnp.float32)
        o_ref[...] = acc

    return pl.pallas_call(
        body,
        grid=(AP // rb,),
        in_specs=[
            pl.BlockSpec((rb, HP), lambda i: (i, 0)),
            pl.BlockSpec((rb, HP), lambda i: (i, 0)),
            pl.BlockSpec((rb, HP), lambda i: (i, 0)),
            pl.BlockSpec((HP, HP), lambda i: (0, 0)),
            pl.BlockSpec((HP, HP), lambda i: (0, 0)),
            pl.BlockSpec((HP, HP), lambda i: (0, 0)),
        ],
        out_specs=pl.BlockSpec((rb, HP), lambda i: (i, 0)),
        out_shape=jax.ShapeDtypeStruct((AP, HP), jnp.float32),
    )(agg, ma, ia, b0, b1, b2)


def _tc_gru(hid_am, gbias, wihf, bihf, whhf, bhhf, wihb, bibh, whhb, bhhb,
            wo1, wo2, wob):
    def body(hid_ref, gb_ref, wihf_ref, bihf_ref, whhf_ref, bhhf_ref,
             wihb_ref, bibh_ref, whhb_ref, bhhb_ref,
             wo1_ref, wo2_ref, wob_ref, o_ref, hf_buf):
        h0 = hid_ref[:, 0, :]
        for t in range(1, T):
            h0 = jnp.maximum(h0, hid_ref[:, t, :])

        gb = gb_ref[...]

        def gru_step(x, h, wih_ref, bih_ref, whh_ref, bhh_ref):
            gi = jnp.dot(x, wih_ref[...],
                         preferred_element_type=jnp.float32) + bih_ref[...]
            gh = jnp.dot(h, whh_ref[...],
                         preferred_element_type=jnp.float32) + bhh_ref[...]
            r = jax.nn.sigmoid(gi[:, 0:HP] + gh[:, 0:HP])
            z = jax.nn.sigmoid(gi[:, HP:2 * HP] + gh[:, HP:2 * HP])
            n = jnp.tanh(gi[:, 2 * HP:3 * HP] + r * gh[:, 2 * HP:3 * HP])
            return (1.0 - z) * n + z * h

        def fwd_step(t, hf):
            xf = jnp.maximum(hid_ref[:, t, :] + gb, 0.0)
            hf = gru_step(xf, hf, wihf_ref, bihf_ref, whhf_ref, bhhf_ref)
            hf_buf[t] = hf
            return hf

        lax.fori_loop(0, T, fwd_step, h0)

        def bwd_step(i, carry):
            hb, acc = carry
            t = T - 1 - i
            xb = jnp.maximum(hid_ref[:, t, :] + gb, 0.0)
            hb = gru_step(xb, hb, wihb_ref, bibh_ref, whhb_ref, bhhb_ref)
            ah = jnp.dot(hf_buf[t], wo1_ref[...],
                         preferred_element_type=jnp.float32)
            ah += jnp.dot(hb, wo2_ref[...],
                          preferred_element_type=jnp.float32)
            ah = jnp.maximum(ah + wob_ref[...], 0.0)
            return (hb, acc + ah)

        _, acc = lax.fori_loop(0, T, bwd_step,
                               (h0, jnp.zeros((NM, HP), jnp.float32)))
        o_ref[...] = acc * (1.0 / T)

    return pl.pallas_call(
        body,
        out_shape=jax.ShapeDtypeStruct((NM, HP), jnp.float32),
        scratch_shapes=[
            pltpu.VMEM((T, NM, HP), jnp.float32),
        ],
    )(hid_am, gbias, wihf, bihf, whhf, bhhf, wihb, bibh, whhb, bhhb,
      wo1, wo2, wob)


def _pad_wt(w):
    return jnp.pad(w.T, ((0, 0), (0, HP - H)))


def _pad_sq(w):
    return jnp.pad(w.T, ((0, HP - H), (0, HP - H)))


def _pad_gates(w):
    wt = w.T
    out = jnp.zeros((HP, G3), jnp.float32)
    for g in range(3):
        out = lax.dynamic_update_slice(
            out, jnp.pad(wt[:, g * H:(g + 1) * H], ((0, HP - H), (0, 0))),
            (0, g * HP))
    return out


def _pad_gate_bias(b):
    out = jnp.zeros((1, G3), jnp.float32)
    for g in range(3):
        out = lax.dynamic_update_slice(out, b[g * H:(g + 1) * H][None, :],
                                       (0, g * HP))
    return out


def kernel(f_atoms, f_bonds, a2b, b2a, b2revb, a_scope, W_i_atom, W_i_bond,
           W_h_0, W_h_1, lr_W, gru_bias, gru_W_ih_f, gru_W_hh_f, gru_b_ih_f,
           gru_b_hh_f, gru_W_ih_b, gru_W_hh_b, gru_b_ih_b, gru_b_hh_b,
           W_o_W, W_o_b):
    a2b_flat = jnp.pad(a2b, ((0, AP - A), (0, 0))).reshape(-1)
    b2a_p = jnp.pad(b2a, (0, BP - B))
    b2revb_p = jnp.pad(b2revb, (0, BP - B))

    wia = _pad_wt(W_i_atom)
    wib = _pad_wt(W_i_bond)
    wh = [_pad_sq(W_h_0), _pad_sq(W_h_1)]
    lrt = lr_W.T
    b0 = jnp.pad(lrt[0:H], ((0, HP - H), (0, HP - H)))
    b1 = jnp.pad(lrt[H:2 * H], ((0, HP - H), (0, HP - H)))
    b2 = jnp.pad(lrt[2 * H:3 * H], ((0, HP - H), (0, HP - H)))
    gb = jnp.pad(gru_bias, (0, HP - H))[None, :]
    wihf = _pad_gates(gru_W_ih_f)
    whhf = _pad_gates(gru_W_hh_f)
    bihf = _pad_gate_bias(gru_b_ih_f)
    bhhf = _pad_gate_bias(gru_b_hh_f)
    wihb = _pad_gates(gru_W_ih_b)
    whhb = _pad_gates(gru_W_hh_b)
    bibh = _pad_gate_bias(gru_b_ih_b)
    bhhb = _pad_gate_bias(gru_b_hh_b)
    wot = W_o_W.T
    wo1 = jnp.pad(wot[0:H], ((0, HP - H), (0, HP - H)))
    wo2 = jnp.pad(wot[H:2 * H], ((0, HP - H), (0, HP - H)))
    wob = jnp.pad(W_o_b, (0, HP - H))[None, :]

    input_atom = _tc_embed(f_atoms, wia, AP, 512)
    input_bond = _tc_embed_t(f_bonds.T, wib, BP, 1024)

    msg_atom = input_atom
    msg_bond = input_bond
    for it in range(2):
        msg_atom, _ = _sc_atom_agg(msg_bond, a2b_flat, msg_atom)
        pre = _sc_bond_gather(msg_atom, msg_bond, b2a_p, b2revb_p)
        msg_bond = _tc_bond_update(pre, wh[it], input_bond)

    _, agg = _sc_atom_agg(msg_bond, a2b_flat, msg_atom)
    hidden = _tc_lr(agg, msg_atom, input_atom, b0, b1, b2)

    hid_am = hidden[1:1 + NM * T].reshape(NM, T, HP)
    mol = _tc_gru(hid_am, gb, wihf, bihf, whhf, bhhf, wihb, bibh, whhb, bhhb,
                  wo1, wo2, wob)
    return mol[:, :H]
